# baseline (speedup 1.0000x reference)
import functools

import jax
import jax.numpy as jnp
from jax import lax
from jax.experimental import pallas as pl
from jax.experimental.pallas import tpu as pltpu
from jax.experimental.pallas import tpu_sc as plsc

EMBED_D = 64
LANES = 128
RELAYOUT_BLK = 8192


def _relayout_body(t_ref, o_ref, o2_ref):
    x = t_ref[...]
    o_ref[:, pl.ds(0, EMBED_D)] = x.T
    o2_ref[...] = x


@functools.lru_cache(maxsize=None)
def _tc_relayout(vocab: int):
    nblk = (vocab + RELAYOUT_BLK - 1) // RELAYOUT_BLK
    return pl.pallas_call(
        _relayout_body,
        grid=(nblk,),
        in_specs=[pl.BlockSpec((EMBED_D, RELAYOUT_BLK), lambda i: (0, i))],
        out_specs=[
            pl.BlockSpec((RELAYOUT_BLK, LANES), lambda i: (i, 0)),
            pl.BlockSpec((EMBED_D, RELAYOUT_BLK), lambda i: (0, i)),
        ],
        out_shape=[
            jax.ShapeDtypeStruct((vocab, LANES), jnp.float32),
            jax.ShapeDtypeStruct((EMBED_D, vocab), jnp.float32),
        ],
    )


@functools.lru_cache(maxsize=None)
def _sc_gather(n_t: int, n_b: int, vocab: int):
    info = plsc.get_sparse_core_info()
    nc, ns = info.num_cores, info.num_subcores
    nw = nc * ns
    tiles_b = n_b // LANES
    tiles_t = n_t // 8
    tpw = (tiles_t * tiles_b) // nw
    bpw = tpw * 8

    mesh = plsc.VectorSubcoreMesh(core_axis_name="c", subcore_axis_name="s")

    @functools.partial(
        pl.kernel,
        mesh=mesh,
        out_type=jax.ShapeDtypeStruct((n_t, n_b, EMBED_D), jnp.float32),
        scratch_types=[
            pltpu.VMEM((2, 8, LANES), jnp.int32),
            pltpu.VMEM((3, LANES, LANES), jnp.float32),
            pltpu.VMEM((2, LANES, EMBED_D), jnp.float32),
            pltpu.SemaphoreType.DMA((3,)),
            pltpu.SemaphoreType.DMA((2,)),
        ],
        compiler_params=pltpu.CompilerParams(use_tc_tiling_on_sc=True),
    )
    def k(idx_hbm, r5_hbm, out_hbm, idx_v, gbuf, cbuf, gsem, wsem):
        wid = lax.axis_index("s") * nc + lax.axis_index("c")

        def fire(kk):
            j = kk // 8
            t_lo = lax.rem(kk, 8)
            jb = lax.rem(j, 2)
            tile = wid * tpw + j

            @pl.when(t_lo == 0)
            def _():
                pltpu.sync_copy(
                    idx_hbm.at[pl.ds((tile // tiles_b) * 8, 8),
                               pl.ds(lax.rem(tile, tiles_b) * LANES, LANES)],
                    idx_v.at[jb])

            pltpu.async_copy(r5_hbm.at[idx_v.at[jb, t_lo]],
                             gbuf.at[lax.rem(kk, 3)],
                             gsem.at[lax.rem(kk, 3)])

        fire(0)
        fire(1)

        def body(kk, carry):
            gb = lax.rem(kk, 3)
            cb = lax.rem(kk, 2)
            j = kk // 8
            t_lo = lax.rem(kk, 8)
            tile = wid * tpw + j

            pltpu.make_async_copy(r5_hbm.at[idx_v.at[lax.rem(j, 2), t_lo]],
                                  gbuf.at[gb], gsem.at[gb]).wait()

            @pl.when(kk + 2 < bpw)
            def _():
                fire(kk + 2)

            @pl.when(kk >= 2)
            def _():
                _wait_write(kk - 2)

            for r in range(LANES):
                for c in range(0, EMBED_D, 16):
                    cbuf[cb, r, pl.ds(c, 16)] = gbuf[gb, r, pl.ds(c, 16)]

            pltpu.async_copy(
                cbuf.at[cb],
                out_hbm.at[(tile // tiles_b) * 8 + t_lo,
                           pl.ds(lax.rem(tile, tiles_b) * LANES, LANES), :],
                wsem.at[cb])
            return carry

        def _wait_write(kk):
            cb = lax.rem(kk, 2)
            j = kk // 8
            t_lo = lax.rem(kk, 8)
            tile = wid * tpw + j
            pltpu.make_async_copy(
                cbuf.at[cb],
                out_hbm.at[(tile // tiles_b) * 8 + t_lo,
                           pl.ds(lax.rem(tile, tiles_b) * LANES, LANES), :],
                wsem.at[cb]).wait()

        lax.fori_loop(0, bpw, body, 0)
        _wait_write(bpw - 2)
        _wait_write(bpw - 1)

    return k


def kernel(inputs, token_embed_weights):
    idx_t = inputs.astype(jnp.int32).T
    tab_t = token_embed_weights.T
    vocab = token_embed_weights.shape[0]
    r5, tcopy_t = _tc_relayout(vocab)(tab_t)
    n_t, n_b = idx_t.shape
    out4 = _sc_gather(n_t, n_b, vocab)(idx_t, r5)
    out = jnp.transpose(out4, (1, 0, 2))
    return out, tcopy_t.T


# speedup vs baseline: 1.1175x; 1.1175x over previous
import functools

import jax
import jax.numpy as jnp
from jax import lax
from jax.experimental import pallas as pl
from jax.experimental.pallas import tpu as pltpu
from jax.experimental.pallas import tpu_sc as plsc

EMBED_D = 64
LANES = 128
RELAYOUT_BLK = 16384


def _relayout_body(t_ref, o_ref, o2_ref):
    x = t_ref[...]
    o_ref[:, pl.ds(0, EMBED_D)] = x.T
    o2_ref[...] = x


@functools.lru_cache(maxsize=None)
def _tc_relayout(vocab: int):
    nblk = (vocab + RELAYOUT_BLK - 1) // RELAYOUT_BLK
    return pl.pallas_call(
        _relayout_body,
        grid=(nblk,),
        in_specs=[pl.BlockSpec((EMBED_D, RELAYOUT_BLK), lambda i: (0, i))],
        out_specs=[
            pl.BlockSpec((RELAYOUT_BLK, LANES), lambda i: (i, 0)),
            pl.BlockSpec((EMBED_D, RELAYOUT_BLK), lambda i: (0, i)),
        ],
        out_shape=[
            jax.ShapeDtypeStruct((vocab, LANES), jnp.float32),
            jax.ShapeDtypeStruct((EMBED_D, vocab), jnp.float32),
        ],
    )


@functools.lru_cache(maxsize=None)
def _sc_gather(n_t: int, n_b: int, vocab: int):
    info = plsc.get_sparse_core_info()
    nc, ns = info.num_cores, info.num_subcores
    nw = nc * ns
    tiles_b = n_b // LANES
    tiles_t = n_t // 8
    tpw = (tiles_t * tiles_b) // nw
    bpw = tpw * 8

    mesh = plsc.VectorSubcoreMesh(core_axis_name="c", subcore_axis_name="s")

    @functools.partial(
        pl.kernel,
        mesh=mesh,
        out_type=jax.ShapeDtypeStruct((n_t, n_b, EMBED_D), jnp.float32),
        scratch_types=[
            pltpu.VMEM((2, 8, LANES), jnp.int32),
            pltpu.VMEM((2, LANES, LANES), jnp.float32),
            pltpu.VMEM((2, LANES, EMBED_D), jnp.float32),
            pltpu.SemaphoreType.DMA((2,)),
            pltpu.SemaphoreType.DMA((2,)),
        ],
        compiler_params=pltpu.CompilerParams(use_tc_tiling_on_sc=True),
    )
    def k(idx_hbm, r5_hbm, out_hbm, idx_v, gbuf, cbuf, gsem, wsem):
        wid = lax.axis_index("s") * nc + lax.axis_index("c")

        def fire(kk):
            j = kk // 8
            t_lo = lax.rem(kk, 8)
            jb = lax.rem(j, 2)
            tile = wid * tpw + j

            @pl.when(t_lo == 0)
            def _():
                pltpu.sync_copy(
                    idx_hbm.at[pl.ds((tile // tiles_b) * 8, 8),
                               pl.ds(lax.rem(tile, tiles_b) * LANES, LANES)],
                    idx_v.at[jb])

            pltpu.async_copy(r5_hbm.at[idx_v.at[jb, t_lo]],
                             gbuf.at[lax.rem(kk, 2)],
                             gsem.at[lax.rem(kk, 2)])

        fire(0)

        def body(kk, carry):
            b = lax.rem(kk, 2)
            j = kk // 8
            t_lo = lax.rem(kk, 8)
            tile = wid * tpw + j

            @pl.when(kk + 1 < bpw)
            def _():
                fire(kk + 1)

            pltpu.make_async_copy(r5_hbm.at[idx_v.at[lax.rem(j, 2), t_lo]],
                                  gbuf.at[b], gsem.at[b]).wait()

            @pl.when(kk >= 2)
            def _():
                _wait_write(kk - 2)

            for r in range(LANES):
                for c in range(0, EMBED_D, 16):
                    cbuf[b, r, pl.ds(c, 16)] = gbuf[b, r, pl.ds(c, 16)]

            pltpu.async_copy(
                cbuf.at[b],
                out_hbm.at[(tile // tiles_b) * 8 + t_lo,
                           pl.ds(lax.rem(tile, tiles_b) * LANES, LANES), :],
                wsem.at[b])
            return carry

        def _wait_write(kk):
            b = lax.rem(kk, 2)
            j = kk // 8
            t_lo = lax.rem(kk, 8)
            tile = wid * tpw + j
            pltpu.make_async_copy(
                cbuf.at[b],
                out_hbm.at[(tile // tiles_b) * 8 + t_lo,
                           pl.ds(lax.rem(tile, tiles_b) * LANES, LANES), :],
                wsem.at[b]).wait()

        lax.fori_loop(0, bpw, body, 0)
        _wait_write(bpw - 2)
        _wait_write(bpw - 1)

    return k


def kernel(inputs, token_embed_weights):
    idx_t = inputs.astype(jnp.int32).T
    tab_t = token_embed_weights.T
    vocab = token_embed_weights.shape[0]
    r5, tcopy_t = _tc_relayout(vocab)(tab_t)
    n_t, n_b = idx_t.shape
    out4 = _sc_gather(n_t, n_b, vocab)(idx_t, r5)
    out = jnp.transpose(out4, (1, 0, 2))
    return out, tcopy_t.T
